# baseline (speedup 1.0000x reference)
import functools

import jax
import jax.numpy as jnp
from jax import lax
from jax.experimental import pallas as pl
from jax.experimental.pallas import tpu as pltpu
from jax.experimental.pallas import tpu_sc as plsc

N = 10000
E = 320000
D = 128
NC = 2
NS = 16
NW = NC * NS
C = 112
WCH = 90
EPAD = NW * WCH * C
NPAD = 10240
NPT = NPAD // NS


LAG = 8


def _sc_degrees(edges_hbm, ones_hbm, zeros_hbm,
                odeg_out, ideg_out,
                odeg_sh, ideg_sh, ones_v, sidx_v, didx_v, osem, isem):
    c = lax.axis_index("c")
    s = lax.axis_index("s")
    wid = c * NS + s
    row0 = pl.multiple_of(s * NPT, 8)
    pltpu.sync_copy(zeros_hbm, odeg_sh.at[pl.ds(row0, NPT)])
    pltpu.sync_copy(zeros_hbm, ideg_sh.at[pl.ds(row0, NPT)])
    pltpu.sync_copy(ones_hbm, ones_v)
    pltpu.sync_copy(edges_hbm.at[0, wid], sidx_v)
    pltpu.sync_copy(edges_hbm.at[1, wid], didx_v)
    plsc.subcore_barrier()

    def fire(j, carry):
        pltpu.async_copy(ones_v, odeg_sh.at[sidx_v.at[j]], osem, add=True)
        pltpu.async_copy(ones_v, ideg_sh.at[didx_v.at[j]], isem, add=True)

        @pl.when(j >= LAG)
        def _():
            jw = j - LAG
            pltpu.make_async_copy(ones_v, odeg_sh.at[sidx_v.at[jw]],
                                  osem).wait()
            pltpu.make_async_copy(ones_v, ideg_sh.at[didx_v.at[jw]],
                                  isem).wait()

        return carry

    lax.fori_loop(0, WCH, fire, 0)

    def drain(j, carry):
        jw = WCH - LAG + j
        pltpu.make_async_copy(ones_v, odeg_sh.at[sidx_v.at[jw]], osem).wait()
        pltpu.make_async_copy(ones_v, ideg_sh.at[didx_v.at[jw]], isem).wait()
        return carry

    lax.fori_loop(0, LAG, drain, 0)
    plsc.subcore_barrier()

    pltpu.sync_copy(odeg_sh.at[pl.ds(row0, NPT)],
                    odeg_out.at[c, pl.ds(row0, NPT)])
    pltpu.sync_copy(ideg_sh.at[pl.ds(row0, NPT)],
                    ideg_out.at[c, pl.ds(row0, NPT)])


def _sc_aggregate(h_hbm, edges_hbm, zeros_hbm,
                  agg_out,
                  agg_sh, sidx_v, didx_v, r0_v, r1_v,
                  gsem0, gsem1):
    c = lax.axis_index("c")
    s = lax.axis_index("s")
    wid = c * NS + s
    row0 = pl.multiple_of(s * NPT, 8)
    pltpu.sync_copy(zeros_hbm, agg_sh.at[pl.ds(row0, NPT)])
    pltpu.sync_copy(edges_hbm.at[0, wid], sidx_v)
    pltpu.sync_copy(edges_hbm.at[1, wid], didx_v)
    plsc.subcore_barrier()

    pltpu.make_async_copy(h_hbm.at[sidx_v.at[0]], r0_v, gsem0).start()

    def body(i, carry):
        j0 = pl.multiple_of(2 * i, 2)
        j1 = j0 + 1
        pltpu.make_async_copy(h_hbm.at[sidx_v.at[j1]], r1_v, gsem1).start()
        pltpu.make_async_copy(h_hbm.at[sidx_v.at[j0]], r0_v, gsem0).wait()
        pltpu.sync_copy(r0_v, agg_sh.at[didx_v.at[j0]], add=True)

        @pl.when(j1 + 1 < WCH)
        def _():
            pltpu.make_async_copy(h_hbm.at[sidx_v.at[j1 + 1]], r0_v,
                                  gsem0).start()

        pltpu.make_async_copy(h_hbm.at[sidx_v.at[j1]], r1_v, gsem1).wait()
        pltpu.sync_copy(r1_v, agg_sh.at[didx_v.at[j1]], add=True)
        return carry

    lax.fori_loop(0, WCH // 2, body, 0)
    plsc.subcore_barrier()

    pltpu.sync_copy(agg_sh.at[pl.ds(row0, NPT)],
                    agg_out.at[c, pl.ds(row0, NPT)])


@functools.lru_cache(maxsize=None)
def _sc_kernels():
    mesh = plsc.VectorSubcoreMesh(core_axis_name="c", subcore_axis_name="s",
                                  num_cores=NC, num_subcores=NS)
    degrees = pl.kernel(
        _sc_degrees,
        out_type=(
            jax.ShapeDtypeStruct((NC, NPAD, 16), jnp.float32),
            jax.ShapeDtypeStruct((NC, NPAD, 16), jnp.float32),
        ),
        mesh=mesh,
        scratch_types=[
            pltpu.VMEM_SHARED((NPAD, 16), jnp.float32),
            pltpu.VMEM_SHARED((NPAD, 16), jnp.float32),
            pltpu.VMEM((C, 16), jnp.float32),
            pltpu.VMEM((WCH, C), jnp.int32),
            pltpu.VMEM((WCH, C), jnp.int32),
            pltpu.SemaphoreType.DMA,
            pltpu.SemaphoreType.DMA,
        ],
        compiler_params=pltpu.CompilerParams(use_tc_tiling_on_sc=False),
    )
    aggregate = pl.kernel(
        _sc_aggregate,
        out_type=jax.ShapeDtypeStruct((NC, NPAD, D), jnp.float32),
        mesh=mesh,
        scratch_types=[
            pltpu.VMEM_SHARED((NPAD, D), jnp.float32),
            pltpu.VMEM((WCH, C), jnp.int32),
            pltpu.VMEM((WCH, C), jnp.int32),
            pltpu.VMEM((C, D), jnp.float32),
            pltpu.VMEM((C, D), jnp.float32),
            pltpu.SemaphoreType.DMA,
            pltpu.SemaphoreType.DMA,
        ],
        compiler_params=pltpu.CompilerParams(use_tc_tiling_on_sc=False),
    )
    return degrees, aggregate


def _tc_scale_body(x_ref, odeg_ref, h_ref):
    od = (lax.slice(odeg_ref[0], (0, 0), (N, 1))
          + lax.slice(odeg_ref[1], (0, 0), (N, 1)))
    onorm = lax.rsqrt(jnp.where(od > 0.0, od, 1.0))
    h = x_ref[...] * onorm
    h_ref[...] = jnp.concatenate(
        [h, jnp.zeros((NPAD - N, D), jnp.float32)], axis=0)


def _tc_final_body(agg_ref, ideg_ref, x_ref, w_ref, b_ref, g_ref, bt_ref,
                   out_ref):
    agg = lax.slice(agg_ref[0], (0, 0), (N, D)) + lax.slice(
        agg_ref[1], (0, 0), (N, D))
    idg = (lax.slice(ideg_ref[0], (0, 0), (N, 1))
           + lax.slice(ideg_ref[1], (0, 0), (N, 1)))
    inorm = lax.rsqrt(jnp.where(idg > 0.0, idg, 1.0))
    h = jnp.dot(agg * inorm, w_ref[...],
                preferred_element_type=jnp.float32) + b_ref[...]
    mean = jnp.mean(h, axis=0, keepdims=True)
    ctr = h - mean
    var = jnp.mean(ctr * ctr, axis=0, keepdims=True)
    hn = ctr * lax.rsqrt(var + 1e-5) * g_ref[...] + bt_ref[...]
    out_ref[...] = jnp.maximum(hn, 0.0) + x_ref[...]


_tc_scale = pl.pallas_call(
    _tc_scale_body,
    out_shape=jax.ShapeDtypeStruct((NPAD, D), jnp.float32),
)

_tc_final = pl.pallas_call(
    _tc_final_body,
    out_shape=jax.ShapeDtypeStruct((N, D), jnp.float32),
)


def kernel(x, edge_index, W, b, gamma, beta):
    sc_degrees, sc_aggregate = _sc_kernels()
    ei = edge_index.astype(jnp.int32)
    pad = jnp.full((2, EPAD - E), NPAD - 1, jnp.int32)
    ei4 = jnp.concatenate([ei, pad], axis=1).reshape(2, NW, WCH, C)
    ones16 = jnp.ones((C, 16), jnp.float32)
    zeros16 = jnp.zeros((NPT, 16), jnp.float32)
    zerosd = jnp.zeros((NPT, D), jnp.float32)

    odeg, ideg = sc_degrees(ei4, ones16, zeros16)
    h = _tc_scale(x, odeg)
    aggp = sc_aggregate(h, ei4, zerosd)
    out = _tc_final(aggp, ideg, x, W, b.reshape(1, D),
                    gamma.reshape(1, D), beta.reshape(1, D))
    return out


# speedup vs baseline: 1.4939x; 1.4939x over previous
import functools

import jax
import jax.numpy as jnp
from jax import lax
from jax.experimental import pallas as pl
from jax.experimental.pallas import tpu as pltpu
from jax.experimental.pallas import tpu_sc as plsc

N = 10000
E = 320000
D = 128
NC = 2
NS = 16
NW = NC * NS
C = 112
WCH = 90
EPAD = NW * WCH * C
NPAD = 10240
NPT = NPAD // NS


LAG = 8


def _sc_degrees(edges_hbm, ones_hbm, zeros_hbm,
                odeg_out, ideg_out,
                odeg_sh, ideg_sh, ones_v, sidx_v, didx_v, osem, isem):
    c = lax.axis_index("c")
    s = lax.axis_index("s")
    wid = c * NS + s
    row0 = pl.multiple_of(s * NPT, 8)
    pltpu.sync_copy(zeros_hbm, odeg_sh.at[pl.ds(row0, NPT)])
    pltpu.sync_copy(zeros_hbm, ideg_sh.at[pl.ds(row0, NPT)])
    pltpu.sync_copy(ones_hbm, ones_v)
    pltpu.sync_copy(edges_hbm.at[0, wid], sidx_v)
    pltpu.sync_copy(edges_hbm.at[1, wid], didx_v)
    plsc.subcore_barrier()

    def fire(j, carry):
        pltpu.async_copy(ones_v, odeg_sh.at[sidx_v.at[j]], osem, add=True)
        pltpu.async_copy(ones_v, ideg_sh.at[didx_v.at[j]], isem, add=True)

        @pl.when(j >= LAG)
        def _():
            jw = j - LAG
            pltpu.make_async_copy(ones_v, odeg_sh.at[sidx_v.at[jw]],
                                  osem).wait()
            pltpu.make_async_copy(ones_v, ideg_sh.at[didx_v.at[jw]],
                                  isem).wait()

        return carry

    lax.fori_loop(0, WCH, fire, 0)

    def drain(j, carry):
        jw = WCH - LAG + j
        pltpu.make_async_copy(ones_v, odeg_sh.at[sidx_v.at[jw]], osem).wait()
        pltpu.make_async_copy(ones_v, ideg_sh.at[didx_v.at[jw]], isem).wait()
        return carry

    lax.fori_loop(0, LAG, drain, 0)
    plsc.subcore_barrier()

    pltpu.sync_copy(odeg_sh.at[pl.ds(row0, NPT)],
                    odeg_out.at[c, pl.ds(row0, NPT)])
    pltpu.sync_copy(ideg_sh.at[pl.ds(row0, NPT)],
                    ideg_out.at[c, pl.ds(row0, NPT)])


def _sc_aggregate(h_hbm, edges_hbm, zeros_hbm,
                  agg_out,
                  agg_sh, sidx_v, didx_v, r0_v, r1_v,
                  gsem0, gsem1):
    c = lax.axis_index("c")
    s = lax.axis_index("s")
    wid = c * NS + s
    row0 = pl.multiple_of(s * NPT, 8)
    pltpu.sync_copy(zeros_hbm, agg_sh.at[pl.ds(row0, NPT)])
    pltpu.sync_copy(edges_hbm.at[0, wid], sidx_v)
    pltpu.sync_copy(edges_hbm.at[1, wid], didx_v)
    plsc.subcore_barrier()

    pltpu.make_async_copy(h_hbm.at[sidx_v.at[0]], r0_v, gsem0).start()

    def body(i, carry):
        j0 = pl.multiple_of(2 * i, 2)
        j1 = j0 + 1
        pltpu.make_async_copy(h_hbm.at[sidx_v.at[j1]], r1_v, gsem1).start()
        pltpu.make_async_copy(h_hbm.at[sidx_v.at[j0]], r0_v, gsem0).wait()
        pltpu.sync_copy(r0_v, agg_sh.at[didx_v.at[j0]], add=True)

        @pl.when(j1 + 1 < WCH)
        def _():
            pltpu.make_async_copy(h_hbm.at[sidx_v.at[j1 + 1]], r0_v,
                                  gsem0).start()

        pltpu.make_async_copy(h_hbm.at[sidx_v.at[j1]], r1_v, gsem1).wait()
        pltpu.sync_copy(r1_v, agg_sh.at[didx_v.at[j1]], add=True)
        return carry

    lax.fori_loop(0, WCH // 2, body, 0)
    plsc.subcore_barrier()

    pltpu.sync_copy(agg_sh.at[pl.ds(row0, NPT)],
                    agg_out.at[c, pl.ds(row0, NPT)])


@functools.lru_cache(maxsize=None)
def _sc_kernels():
    mesh = plsc.VectorSubcoreMesh(core_axis_name="c", subcore_axis_name="s",
                                  num_cores=NC, num_subcores=NS)
    degrees = pl.kernel(
        _sc_degrees,
        out_type=(
            jax.ShapeDtypeStruct((NC, NPAD, 16), jnp.float32),
            jax.ShapeDtypeStruct((NC, NPAD, 16), jnp.float32),
        ),
        mesh=mesh,
        scratch_types=[
            pltpu.VMEM_SHARED((NPAD, 16), jnp.float32),
            pltpu.VMEM_SHARED((NPAD, 16), jnp.float32),
            pltpu.VMEM((C, 16), jnp.float32),
            pltpu.VMEM((WCH, C), jnp.int32),
            pltpu.VMEM((WCH, C), jnp.int32),
            pltpu.SemaphoreType.DMA,
            pltpu.SemaphoreType.DMA,
        ],
        compiler_params=pltpu.CompilerParams(use_tc_tiling_on_sc=False),
    )
    aggregate = pl.kernel(
        _sc_aggregate,
        out_type=jax.ShapeDtypeStruct((NC, NPAD, D), jnp.float32),
        mesh=mesh,
        scratch_types=[
            pltpu.VMEM_SHARED((NPAD, D), jnp.float32),
            pltpu.VMEM((WCH, C), jnp.int32),
            pltpu.VMEM((WCH, C), jnp.int32),
            pltpu.VMEM((C, D), jnp.float32),
            pltpu.VMEM((C, D), jnp.float32),
            pltpu.SemaphoreType.DMA,
            pltpu.SemaphoreType.DMA,
        ],
        compiler_params=pltpu.CompilerParams(use_tc_tiling_on_sc=False),
    )
    return degrees, aggregate


def _tc_scale_body(x_ref, odeg_ref, h_ref):
    od = (lax.slice(odeg_ref[0], (0, 0), (N, 1))
          + lax.slice(odeg_ref[1], (0, 0), (N, 1)))
    onorm = lax.rsqrt(jnp.where(od > 0.0, od, 1.0))
    h = x_ref[...] * onorm
    h_ref[...] = jnp.concatenate(
        [h, jnp.zeros((NPAD - N, D), jnp.float32)], axis=0)


def _tc_final_body(agg_ref, ideg_ref, x_ref, w_ref, b_ref, g_ref, bt_ref,
                   out_ref):
    agg = lax.slice(agg_ref[0], (0, 0), (N, D)) + lax.slice(
        agg_ref[1], (0, 0), (N, D))
    idg = (lax.slice(ideg_ref[0], (0, 0), (N, 1))
           + lax.slice(ideg_ref[1], (0, 0), (N, 1)))
    inorm = lax.rsqrt(jnp.where(idg > 0.0, idg, 1.0))
    h = jnp.dot(agg * inorm, w_ref[...],
                preferred_element_type=jnp.float32) + b_ref[...]
    mean = jnp.mean(h, axis=0, keepdims=True)
    ctr = h - mean
    var = jnp.mean(ctr * ctr, axis=0, keepdims=True)
    hn = ctr * lax.rsqrt(var + 1e-5) * g_ref[...] + bt_ref[...]
    out_ref[...] = jnp.maximum(hn, 0.0) + x_ref[...]


_tc_scale = pl.pallas_call(
    _tc_scale_body,
    out_shape=jax.ShapeDtypeStruct((NPAD, D), jnp.float32),
)

_tc_final = pl.pallas_call(
    _tc_final_body,
    out_shape=jax.ShapeDtypeStruct((N, D), jnp.float32),
)


def kernel(x, edge_index, W, b, gamma, beta):
    sc_degrees, sc_aggregate = _sc_kernels()
    ei = edge_index.astype(jnp.int32)
    pad = jnp.tile(N + (jnp.arange(EPAD - E, dtype=jnp.int32) % (NPAD - N)),
                   (2, 1))
    ei4 = jnp.concatenate([ei, pad], axis=1).reshape(2, NW, WCH, C)
    ones16 = jnp.ones((C, 16), jnp.float32)
    zeros16 = jnp.zeros((NPT, 16), jnp.float32)
    zerosd = jnp.zeros((NPT, D), jnp.float32)

    odeg, ideg = sc_degrees(ei4, ones16, zeros16)
    h = _tc_scale(x, odeg)
    aggp = sc_aggregate(h, ei4, zerosd)
    out = _tc_final(aggp, ideg, x, W, b.reshape(1, D),
                    gamma.reshape(1, D), beta.reshape(1, D))
    return out


# speedup vs baseline: 1.5192x; 1.0170x over previous
import functools

import jax
import jax.numpy as jnp
from jax import lax
from jax.experimental import pallas as pl
from jax.experimental.pallas import tpu as pltpu
from jax.experimental.pallas import tpu_sc as plsc

N = 10000
E = 320000
D = 128
NC = 2
NS = 16
NW = NC * NS
C = 112
WCH = 90
EPAD = NW * WCH * C
NPAD = 10240
NPT = NPAD // NS


LAG = 8


def _sc_degrees(edges_hbm, ones_hbm, zeros_hbm,
                odeg_out, ideg_out,
                odeg_sh, ideg_sh, ones_v, sidx_v, didx_v, osem, isem,
                zsem):
    c = lax.axis_index("c")
    s = lax.axis_index("s")
    wid = c * NS + s
    row0 = pl.multiple_of(s * NPT, 8)
    pltpu.async_copy(zeros_hbm, odeg_sh.at[pl.ds(row0, NPT)], zsem)
    pltpu.async_copy(zeros_hbm, ideg_sh.at[pl.ds(row0, NPT)], zsem)
    pltpu.sync_copy(ones_hbm, ones_v)
    pltpu.sync_copy(edges_hbm.at[0, wid], sidx_v)
    pltpu.sync_copy(edges_hbm.at[1, wid], didx_v)
    pltpu.make_async_copy(zeros_hbm, odeg_sh.at[pl.ds(row0, NPT)], zsem).wait()
    pltpu.make_async_copy(zeros_hbm, ideg_sh.at[pl.ds(row0, NPT)], zsem).wait()
    plsc.subcore_barrier()

    def fire(j, carry):
        pltpu.async_copy(ones_v, odeg_sh.at[sidx_v.at[j]], osem, add=True)
        pltpu.async_copy(ones_v, ideg_sh.at[didx_v.at[j]], isem, add=True)

        @pl.when(j >= LAG)
        def _():
            jw = j - LAG
            pltpu.make_async_copy(ones_v, odeg_sh.at[sidx_v.at[jw]],
                                  osem).wait()
            pltpu.make_async_copy(ones_v, ideg_sh.at[didx_v.at[jw]],
                                  isem).wait()

        return carry

    lax.fori_loop(0, WCH, fire, 0)

    def drain(j, carry):
        jw = WCH - LAG + j
        pltpu.make_async_copy(ones_v, odeg_sh.at[sidx_v.at[jw]], osem).wait()
        pltpu.make_async_copy(ones_v, ideg_sh.at[didx_v.at[jw]], isem).wait()
        return carry

    lax.fori_loop(0, LAG, drain, 0)
    plsc.subcore_barrier()

    pltpu.sync_copy(odeg_sh.at[pl.ds(row0, NPT)],
                    odeg_out.at[c, pl.ds(row0, NPT)])
    pltpu.sync_copy(ideg_sh.at[pl.ds(row0, NPT)],
                    ideg_out.at[c, pl.ds(row0, NPT)])


def _sc_aggregate(h_hbm, edges_hbm, zeros_hbm,
                  agg_out,
                  agg_sh, sidx_v, didx_v, r0_v, r1_v,
                  gsem0, gsem1, zsem):
    c = lax.axis_index("c")
    s = lax.axis_index("s")
    wid = c * NS + s
    row0 = pl.multiple_of(s * NPT, 8)
    pltpu.async_copy(zeros_hbm, agg_sh.at[pl.ds(row0, NPT)], zsem)
    pltpu.sync_copy(edges_hbm.at[0, wid], sidx_v)
    pltpu.sync_copy(edges_hbm.at[1, wid], didx_v)
    pltpu.make_async_copy(zeros_hbm, agg_sh.at[pl.ds(row0, NPT)], zsem).wait()
    plsc.subcore_barrier()

    pltpu.make_async_copy(h_hbm.at[sidx_v.at[0]], r0_v, gsem0).start()

    def body(i, carry):
        j0 = pl.multiple_of(2 * i, 2)
        j1 = j0 + 1
        pltpu.make_async_copy(h_hbm.at[sidx_v.at[j1]], r1_v, gsem1).start()
        pltpu.make_async_copy(h_hbm.at[sidx_v.at[j0]], r0_v, gsem0).wait()
        pltpu.sync_copy(r0_v, agg_sh.at[didx_v.at[j0]], add=True)

        @pl.when(j1 + 1 < WCH)
        def _():
            pltpu.make_async_copy(h_hbm.at[sidx_v.at[j1 + 1]], r0_v,
                                  gsem0).start()

        pltpu.make_async_copy(h_hbm.at[sidx_v.at[j1]], r1_v, gsem1).wait()
        pltpu.sync_copy(r1_v, agg_sh.at[didx_v.at[j1]], add=True)
        return carry

    lax.fori_loop(0, WCH // 2, body, 0)
    plsc.subcore_barrier()

    pltpu.sync_copy(agg_sh.at[pl.ds(row0, NPT)],
                    agg_out.at[c, pl.ds(row0, NPT)])


@functools.lru_cache(maxsize=None)
def _sc_kernels():
    mesh = plsc.VectorSubcoreMesh(core_axis_name="c", subcore_axis_name="s",
                                  num_cores=NC, num_subcores=NS)
    degrees = pl.kernel(
        _sc_degrees,
        out_type=(
            jax.ShapeDtypeStruct((NC, NPAD, 16), jnp.float32),
            jax.ShapeDtypeStruct((NC, NPAD, 16), jnp.float32),
        ),
        mesh=mesh,
        scratch_types=[
            pltpu.VMEM_SHARED((NPAD, 16), jnp.float32),
            pltpu.VMEM_SHARED((NPAD, 16), jnp.float32),
            pltpu.VMEM((C, 16), jnp.float32),
            pltpu.VMEM((WCH, C), jnp.int32),
            pltpu.VMEM((WCH, C), jnp.int32),
            pltpu.SemaphoreType.DMA,
            pltpu.SemaphoreType.DMA,
            pltpu.SemaphoreType.DMA,
        ],
        compiler_params=pltpu.CompilerParams(use_tc_tiling_on_sc=False),
    )
    aggregate = pl.kernel(
        _sc_aggregate,
        out_type=jax.ShapeDtypeStruct((NC, NPAD, D), jnp.float32),
        mesh=mesh,
        scratch_types=[
            pltpu.VMEM_SHARED((NPAD, D), jnp.float32),
            pltpu.VMEM((WCH, C), jnp.int32),
            pltpu.VMEM((WCH, C), jnp.int32),
            pltpu.VMEM((C, D), jnp.float32),
            pltpu.VMEM((C, D), jnp.float32),
            pltpu.SemaphoreType.DMA,
            pltpu.SemaphoreType.DMA,
            pltpu.SemaphoreType.DMA,
        ],
        compiler_params=pltpu.CompilerParams(use_tc_tiling_on_sc=False),
    )
    return degrees, aggregate


def _tc_scale_body(x_ref, odeg_ref, h_ref):
    od = (lax.slice(odeg_ref[0], (0, 0), (N, 1))
          + lax.slice(odeg_ref[1], (0, 0), (N, 1)))
    onorm = lax.rsqrt(jnp.where(od > 0.0, od, 1.0))
    h = x_ref[...] * onorm
    h_ref[...] = jnp.concatenate(
        [h, jnp.zeros((NPAD - N, D), jnp.float32)], axis=0)


def _tc_final_body(agg_ref, ideg_ref, x_ref, w_ref, b_ref, g_ref, bt_ref,
                   out_ref):
    agg = lax.slice(agg_ref[0], (0, 0), (N, D)) + lax.slice(
        agg_ref[1], (0, 0), (N, D))
    idg = (lax.slice(ideg_ref[0], (0, 0), (N, 1))
           + lax.slice(ideg_ref[1], (0, 0), (N, 1)))
    inorm = lax.rsqrt(jnp.where(idg > 0.0, idg, 1.0))
    h = jnp.dot(agg * inorm, w_ref[...],
                preferred_element_type=jnp.float32) + b_ref[...]
    mean = jnp.mean(h, axis=0, keepdims=True)
    ctr = h - mean
    var = jnp.mean(ctr * ctr, axis=0, keepdims=True)
    hn = ctr * lax.rsqrt(var + 1e-5) * g_ref[...] + bt_ref[...]
    out_ref[...] = jnp.maximum(hn, 0.0) + x_ref[...]


_tc_scale = pl.pallas_call(
    _tc_scale_body,
    out_shape=jax.ShapeDtypeStruct((NPAD, D), jnp.float32),
)

_tc_final = pl.pallas_call(
    _tc_final_body,
    out_shape=jax.ShapeDtypeStruct((N, D), jnp.float32),
)


def kernel(x, edge_index, W, b, gamma, beta):
    sc_degrees, sc_aggregate = _sc_kernels()
    ei = edge_index.astype(jnp.int32)
    pad = jnp.tile(N + (jnp.arange(EPAD - E, dtype=jnp.int32) % (NPAD - N)),
                   (2, 1))
    ei4 = jnp.concatenate([ei, pad], axis=1).reshape(2, NW, WCH, C)
    ones16 = jnp.ones((C, 16), jnp.float32)
    zeros16 = jnp.zeros((NPT, 16), jnp.float32)
    zerosd = jnp.zeros((NPT, D), jnp.float32)

    odeg, ideg = sc_degrees(ei4, ones16, zeros16)
    h = _tc_scale(x, odeg)
    aggp = sc_aggregate(h, ei4, zerosd)
    out = _tc_final(aggp, ideg, x, W, b.reshape(1, D),
                    gamma.reshape(1, D), beta.reshape(1, D))
    return out
